# Initial kernel scaffold; baseline (speedup 1.0000x reference)
import jax
import jax.numpy as jnp
from jax.experimental import pallas as pl


def kernel(x, edge_index, W1a, b1a, W1b, b1b, g1, be1, W2a, b2a, W2b, b2b, g2, be2, Wf1, bf1, Wf2, bf2):
    raise NotImplementedError("write your pallas kernel here")


# speedup vs baseline: 9.4748x; 9.4748x over previous
import functools

import jax
import jax.numpy as jnp
from jax import lax
from jax.experimental import pallas as pl
from jax.experimental.pallas import tpu as pltpu
from jax.experimental.pallas import tpu_sc as plsc

N = 10000
E = 320000
D = 128
H = 32
C = 40

NC = 2
NS = 16
NW = NC * NS
EB = 128
EPW = E // NW
CHUNKS = -(-EPW // EB)
EPW_PAD = CHUNKS * EB
NP = 10112
RPT = NP // NS

BN_ROWS = 2000
GRID = N // BN_ROWS


def _sc_agg_body(u_hbm, src_hbm, dst_hbm, zeros_hbm, out_hbm,
                 sidx_v, didx_v, rows_v, stage_v, acc_sh, sem):
    c = lax.axis_index("c")
    s = lax.axis_index("s")
    wid = c * NS + s

    pltpu.sync_copy(zeros_hbm.at[pl.ds(s * RPT, RPT)], stage_v)
    pltpu.sync_copy(stage_v, acc_sh.at[pl.ds(s * RPT, RPT)])

    pltpu.sync_copy(src_hbm.at[wid], sidx_v)
    pltpu.sync_copy(dst_hbm.at[wid], didx_v)
    plsc.subcore_barrier()

    def body(j, carry):
        pltpu.async_copy(u_hbm.at[sidx_v.at[j]], rows_v, sem).wait()
        pltpu.sync_copy(rows_v, acc_sh.at[didx_v.at[j]], add=True)
        return carry

    lax.fori_loop(0, CHUNKS, body, 0)
    plsc.subcore_barrier()

    pltpu.sync_copy(acc_sh.at[pl.ds(s * RPT, RPT)], stage_v)
    pltpu.sync_copy(stage_v, out_hbm.at[c, pl.ds(s * RPT, RPT)])


_sc_agg = functools.partial(
    pl.kernel,
    out_type=jax.ShapeDtypeStruct((NC, NP, H), jnp.float32),
    mesh=plsc.VectorSubcoreMesh(core_axis_name="c", subcore_axis_name="s",
                                num_cores=NC, num_subcores=NS),
    scratch_types=[
        pltpu.VMEM((CHUNKS, EB), jnp.int32),
        pltpu.VMEM((CHUNKS, EB), jnp.int32),
        pltpu.VMEM((EB, H), jnp.float32),
        pltpu.VMEM((RPT, H), jnp.float32),
        pltpu.VMEM_SHARED((NP, H), jnp.float32),
        pltpu.SemaphoreType.DMA,
    ],
    compiler_params=pltpu.CompilerParams(use_tc_tiling_on_sc=False),
)(_sc_agg_body)


def _mm_body(x_ref, w_ref, o_ref):
    o_ref[...] = jnp.dot(x_ref[...], w_ref[...],
                         preferred_element_type=jnp.float32, precision=jax.lax.Precision.HIGHEST)


def _conv_post_body(u_ref, a0_ref, a1_ref, ba_ref, wb_ref, bb_ref,
                    h_ref, s_ref, s2_ref):
    z = jnp.maximum(u_ref[...] + a0_ref[...] + a1_ref[...] + ba_ref[...], 0.0)
    h = jnp.dot(z, wb_ref[...], preferred_element_type=jnp.float32, precision=jax.lax.Precision.HIGHEST) + bb_ref[...]
    h_ref[...] = h

    @pl.when(pl.program_id(0) == 0)
    def _():
        s_ref[...] = jnp.zeros_like(s_ref)
        s2_ref[...] = jnp.zeros_like(s2_ref)

    hr = h.reshape(BN_ROWS // 8, 8, H)
    s_ref[...] += jnp.sum(hr, axis=0)
    s2_ref[...] += jnp.sum(hr * hr, axis=0)


def _bn_mm_body(h_ref, s_ref, s2_ref, g_ref, be_ref, w_ref, o_ref):
    sm = jnp.sum(s_ref[...], axis=0, keepdims=True)
    sq = jnp.sum(s2_ref[...], axis=0, keepdims=True)
    m = sm / N
    var = sq / N - m * m
    scale = g_ref[...] * jax.lax.rsqrt(var + 1e-5)
    shift = be_ref[...] - m * scale
    hn = h_ref[...] * scale + shift
    o_ref[...] = jnp.dot(hn, w_ref[...], preferred_element_type=jnp.float32, precision=jax.lax.Precision.HIGHEST)


def _head_body(h_ref, s_ref, s2_ref, g_ref, be_ref, wf1_ref, bf1_ref,
               wf2_ref, bf2_ref, o_ref):
    sm = jnp.sum(s_ref[...], axis=0, keepdims=True)
    sq = jnp.sum(s2_ref[...], axis=0, keepdims=True)
    m = sm / N
    var = sq / N - m * m
    scale = g_ref[...] * jax.lax.rsqrt(var + 1e-5)
    shift = be_ref[...] - m * scale
    hn = h_ref[...] * scale + shift
    f = jnp.maximum(
        jnp.dot(hn, wf1_ref[...], preferred_element_type=jnp.float32, precision=jax.lax.Precision.HIGHEST)
        + bf1_ref[...], 0.0)
    o_ref[...] = jnp.dot(f, wf2_ref[...],
                         preferred_element_type=jnp.float32, precision=jax.lax.Precision.HIGHEST) + bf2_ref[...]


def _row_spec(width):
    return pl.BlockSpec((BN_ROWS, width), lambda i: (i, 0))


def _full_spec(shape):
    return pl.BlockSpec(shape, lambda i: tuple(0 for _ in shape))


def _mm(x, w, in_width, out_width):
    return pl.pallas_call(
        _mm_body,
        grid=(GRID,),
        in_specs=[_row_spec(in_width), _full_spec(w.shape)],
        out_specs=_row_spec(out_width),
        out_shape=jax.ShapeDtypeStruct((N, out_width), jnp.float32),
    )(x, w)


def _conv_post(u, agg, ba, wb, bb):
    return pl.pallas_call(
        _conv_post_body,
        grid=(GRID,),
        in_specs=[_row_spec(H), _row_spec(H), _row_spec(H),
                  _full_spec((1, H)), _full_spec((H, H)), _full_spec((1, H))],
        out_specs=[_row_spec(H), _full_spec((8, H)), _full_spec((8, H))],
        out_shape=[jax.ShapeDtypeStruct((N, H), jnp.float32),
                   jax.ShapeDtypeStruct((8, H), jnp.float32),
                   jax.ShapeDtypeStruct((8, H), jnp.float32)],
    )(u, agg[0], agg[1], ba.reshape(1, H), wb, bb.reshape(1, H))


def _bn_mm(h, s, s2, g, be, w):
    return pl.pallas_call(
        _bn_mm_body,
        grid=(GRID,),
        in_specs=[_row_spec(H), _full_spec((8, H)), _full_spec((8, H)),
                  _full_spec((1, H)), _full_spec((1, H)), _full_spec((H, H))],
        out_specs=_row_spec(H),
        out_shape=jax.ShapeDtypeStruct((N, H), jnp.float32),
    )(h, s, s2, g.reshape(1, H), be.reshape(1, H), w)


def _head(h, s, s2, g, be, wf1, bf1, wf2, bf2):
    return pl.pallas_call(
        _head_body,
        grid=(GRID,),
        in_specs=[_row_spec(H), _full_spec((8, H)), _full_spec((8, H)),
                  _full_spec((1, H)), _full_spec((1, H)),
                  _full_spec((H, H)), _full_spec((1, H)),
                  _full_spec((H, C)), _full_spec((1, C))],
        out_specs=_row_spec(C),
        out_shape=jax.ShapeDtypeStruct((N, C), jnp.float32),
    )(h, s, s2, g.reshape(1, H), be.reshape(1, H),
      wf1, bf1.reshape(1, H), wf2, bf2.reshape(1, C))


def kernel(x, edge_index, W1a, b1a, W1b, b1b, g1, be1,
           W2a, b2a, W2b, b2b, g2, be2, Wf1, bf1, Wf2, bf2):
    src = edge_index[0]
    dst = edge_index[1]
    pad = NW * EPW_PAD - E
    src_p = jnp.concatenate(
        [src, jnp.zeros((pad,), jnp.int32)]).reshape(NW, CHUNKS, EB)
    dst_p = jnp.concatenate(
        [dst, jnp.full((pad,), N, jnp.int32)]).reshape(NW, CHUNKS, EB)
    zeros = jnp.zeros((NP, H), jnp.float32)

    u = _mm(x, W1a, D, H)
    agg = _sc_agg(u, src_p, dst_p, zeros)
    h1, s1, s1sq = _conv_post(u, agg, b1a, W1b, b1b)
    v2 = _bn_mm(h1, s1, s1sq, g1, be1, W2a)
    agg2 = _sc_agg(v2, src_p, dst_p, zeros)
    h2, s2, s2sq = _conv_post(v2, agg2, b2a, W2b, b2b)
    return _head(h2, s2, s2sq, g2, be2, Wf1, bf1, Wf2, bf2)


# speedup vs baseline: 10.0901x; 1.0649x over previous
import functools

import jax
import jax.numpy as jnp
from jax import lax
from jax.experimental import pallas as pl
from jax.experimental.pallas import tpu as pltpu
from jax.experimental.pallas import tpu_sc as plsc

N = 10000
E = 320000
D = 128
H = 32
C = 40

NC = 2
NS = 16
NW = NC * NS
EB = 128
EPW = E // NW
CHUNKS = 80
EPW_PAD = CHUNKS * EB
NP = 10112
RPT = NP // NS

BN_ROWS = 2000
GRID = N // BN_ROWS


def _sc_agg_body(u_hbm, src_hbm, dst_hbm, zeros_hbm, out_hbm,
                 sidx_v, didx_v, rows0_v, rows1_v, stage_v, acc_sh,
                 sem0, sem1):
    c = lax.axis_index("c")
    s = lax.axis_index("s")
    wid = c * NS + s

    pltpu.sync_copy(zeros_hbm.at[pl.ds(s * RPT, RPT)], stage_v)
    pltpu.sync_copy(stage_v, acc_sh.at[pl.ds(s * RPT, RPT)])

    pltpu.sync_copy(src_hbm.at[wid], sidx_v)
    pltpu.sync_copy(dst_hbm.at[wid], didx_v)
    plsc.subcore_barrier()

    pltpu.async_copy(u_hbm.at[sidx_v.at[0]], rows0_v, sem0)
    pltpu.async_copy(u_hbm.at[sidx_v.at[1]], rows1_v, sem1)

    def body(g, carry):
        j0 = 2 * g
        pltpu.make_async_copy(u_hbm.at[sidx_v.at[j0]], rows0_v, sem0).wait()
        pltpu.sync_copy(rows0_v, acc_sh.at[didx_v.at[j0]], add=True)

        @pl.when(g + 1 < CHUNKS // 2)
        def _():
            pltpu.async_copy(u_hbm.at[sidx_v.at[j0 + 2]], rows0_v, sem0)

        pltpu.make_async_copy(u_hbm.at[sidx_v.at[j0 + 1]], rows1_v,
                              sem1).wait()
        pltpu.sync_copy(rows1_v, acc_sh.at[didx_v.at[j0 + 1]], add=True)

        @pl.when(g + 1 < CHUNKS // 2)
        def _():
            pltpu.async_copy(u_hbm.at[sidx_v.at[j0 + 3]], rows1_v, sem1)

        return carry

    lax.fori_loop(0, CHUNKS // 2, body, 0)
    plsc.subcore_barrier()

    pltpu.sync_copy(acc_sh.at[pl.ds(s * RPT, RPT)], stage_v)
    pltpu.sync_copy(stage_v, out_hbm.at[c, pl.ds(s * RPT, RPT)])


_sc_agg = functools.partial(
    pl.kernel,
    out_type=jax.ShapeDtypeStruct((NC, NP, H), jnp.float32),
    mesh=plsc.VectorSubcoreMesh(core_axis_name="c", subcore_axis_name="s",
                                num_cores=NC, num_subcores=NS),
    scratch_types=[
        pltpu.VMEM((CHUNKS, EB), jnp.int32),
        pltpu.VMEM((CHUNKS, EB), jnp.int32),
        pltpu.VMEM((EB, H), jnp.float32),
        pltpu.VMEM((EB, H), jnp.float32),
        pltpu.VMEM((RPT, H), jnp.float32),
        pltpu.VMEM_SHARED((NP, H), jnp.float32),
        pltpu.SemaphoreType.DMA,
        pltpu.SemaphoreType.DMA,
    ],
    compiler_params=pltpu.CompilerParams(use_tc_tiling_on_sc=False),
)(_sc_agg_body)


def _mm_body(x_ref, w_ref, o_ref):
    o_ref[...] = jnp.dot(x_ref[...], w_ref[...],
                         preferred_element_type=jnp.float32, precision=jax.lax.Precision.HIGHEST)


def _conv_post_body(u_ref, a0_ref, a1_ref, ba_ref, wb_ref, bb_ref,
                    h_ref, s_ref, s2_ref):
    z = jnp.maximum(u_ref[...] + a0_ref[...] + a1_ref[...] + ba_ref[...], 0.0)
    h = jnp.dot(z, wb_ref[...], preferred_element_type=jnp.float32, precision=jax.lax.Precision.HIGHEST) + bb_ref[...]
    h_ref[...] = h

    @pl.when(pl.program_id(0) == 0)
    def _():
        s_ref[...] = jnp.zeros_like(s_ref)
        s2_ref[...] = jnp.zeros_like(s2_ref)

    hr = h.reshape(BN_ROWS // 8, 8, H)
    s_ref[...] += jnp.sum(hr, axis=0)
    s2_ref[...] += jnp.sum(hr * hr, axis=0)


def _bn_mm_body(h_ref, s_ref, s2_ref, g_ref, be_ref, w_ref, o_ref):
    sm = jnp.sum(s_ref[...], axis=0, keepdims=True)
    sq = jnp.sum(s2_ref[...], axis=0, keepdims=True)
    m = sm / N
    var = sq / N - m * m
    scale = g_ref[...] * jax.lax.rsqrt(var + 1e-5)
    shift = be_ref[...] - m * scale
    hn = h_ref[...] * scale + shift
    o_ref[...] = jnp.dot(hn, w_ref[...], preferred_element_type=jnp.float32, precision=jax.lax.Precision.HIGHEST)


def _head_body(h_ref, s_ref, s2_ref, g_ref, be_ref, wf1_ref, bf1_ref,
               wf2_ref, bf2_ref, o_ref):
    sm = jnp.sum(s_ref[...], axis=0, keepdims=True)
    sq = jnp.sum(s2_ref[...], axis=0, keepdims=True)
    m = sm / N
    var = sq / N - m * m
    scale = g_ref[...] * jax.lax.rsqrt(var + 1e-5)
    shift = be_ref[...] - m * scale
    hn = h_ref[...] * scale + shift
    f = jnp.maximum(
        jnp.dot(hn, wf1_ref[...], preferred_element_type=jnp.float32, precision=jax.lax.Precision.HIGHEST)
        + bf1_ref[...], 0.0)
    o_ref[...] = jnp.dot(f, wf2_ref[...],
                         preferred_element_type=jnp.float32, precision=jax.lax.Precision.HIGHEST) + bf2_ref[...]


def _row_spec(width):
    return pl.BlockSpec((BN_ROWS, width), lambda i: (i, 0))


def _full_spec(shape):
    return pl.BlockSpec(shape, lambda i: tuple(0 for _ in shape))


def _mm(x, w, in_width, out_width):
    return pl.pallas_call(
        _mm_body,
        grid=(GRID,),
        in_specs=[_row_spec(in_width), _full_spec(w.shape)],
        out_specs=_row_spec(out_width),
        out_shape=jax.ShapeDtypeStruct((N, out_width), jnp.float32),
    )(x, w)


def _conv_post(u, agg, ba, wb, bb):
    return pl.pallas_call(
        _conv_post_body,
        grid=(GRID,),
        in_specs=[_row_spec(H), _row_spec(H), _row_spec(H),
                  _full_spec((1, H)), _full_spec((H, H)), _full_spec((1, H))],
        out_specs=[_row_spec(H), _full_spec((8, H)), _full_spec((8, H))],
        out_shape=[jax.ShapeDtypeStruct((N, H), jnp.float32),
                   jax.ShapeDtypeStruct((8, H), jnp.float32),
                   jax.ShapeDtypeStruct((8, H), jnp.float32)],
    )(u, agg[0], agg[1], ba.reshape(1, H), wb, bb.reshape(1, H))


def _bn_mm(h, s, s2, g, be, w):
    return pl.pallas_call(
        _bn_mm_body,
        grid=(GRID,),
        in_specs=[_row_spec(H), _full_spec((8, H)), _full_spec((8, H)),
                  _full_spec((1, H)), _full_spec((1, H)), _full_spec((H, H))],
        out_specs=_row_spec(H),
        out_shape=jax.ShapeDtypeStruct((N, H), jnp.float32),
    )(h, s, s2, g.reshape(1, H), be.reshape(1, H), w)


def _head(h, s, s2, g, be, wf1, bf1, wf2, bf2):
    return pl.pallas_call(
        _head_body,
        grid=(GRID,),
        in_specs=[_row_spec(H), _full_spec((8, H)), _full_spec((8, H)),
                  _full_spec((1, H)), _full_spec((1, H)),
                  _full_spec((H, H)), _full_spec((1, H)),
                  _full_spec((H, C)), _full_spec((1, C))],
        out_specs=_row_spec(C),
        out_shape=jax.ShapeDtypeStruct((N, C), jnp.float32),
    )(h, s, s2, g.reshape(1, H), be.reshape(1, H),
      wf1, bf1.reshape(1, H), wf2, bf2.reshape(1, C))


def kernel(x, edge_index, W1a, b1a, W1b, b1b, g1, be1,
           W2a, b2a, W2b, b2b, g2, be2, Wf1, bf1, Wf2, bf2):
    src = edge_index[0]
    dst = edge_index[1]
    pad = NW * EPW_PAD - E
    src_p = jnp.concatenate(
        [src, jnp.zeros((pad,), jnp.int32)]).reshape(NW, CHUNKS, EB)
    dum = N + jnp.arange(pad, dtype=jnp.int32) % (NP - N)
    dst_p = jnp.concatenate([dst, dum]).reshape(NW, CHUNKS, EB)
    zeros = jnp.zeros((NP, H), jnp.float32)

    u = _mm(x, W1a, D, H)
    agg = _sc_agg(u, src_p, dst_p, zeros)
    h1, s1, s1sq = _conv_post(u, agg, b1a, W1b, b1b)
    v2 = _bn_mm(h1, s1, s1sq, g1, be1, W2a)
    agg2 = _sc_agg(v2, src_p, dst_p, zeros)
    h2, s2, s2sq = _conv_post(v2, agg2, b2a, W2b, b2b)
    return _head(h2, s2, s2sq, g2, be2, Wf1, bf1, Wf2, bf2)
